# Initial kernel scaffold; baseline (speedup 1.0000x reference)
import jax
import jax.numpy as jnp
from jax.experimental import pallas as pl


def kernel(sparse_features, embed_tables, linear_tables, W1, b1, g1, be1, W2, b2, g2, be2, W3, b3, g3, be3, Wo, bo):
    raise NotImplementedError("write your pallas kernel here")


# speedup vs baseline: 1.1354x; 1.1354x over previous
import functools

import jax
import jax.numpy as jnp
from jax import lax
from jax.experimental import pallas as pl
from jax.experimental.pallas import tpu as pltpu
from jax.experimental.pallas import tpu_sc as plsc

_B, _F, _V, _D = 4096, 26, 100000, 16
_NC, _NS = 2, 16
_NW = _NC * _NS
_PW = (_B * _F) // _NW
_CH = 128
_NCH = _PW // _CH


def _gather_body(emb_hbm, lin_hbm, idx_hbm, emb_out, lin_out,
                 idx_v, rows_v, lin_v, sem_e, sem_l):
    wid = lax.axis_index("s") * _NC + lax.axis_index("c")
    base = pl.multiple_of(wid * _PW, _PW)
    pltpu.sync_copy(idx_hbm.at[wid], idx_v)

    def fire(j, _):
        off = pl.multiple_of(j * _CH, _CH)
        pltpu.make_async_copy(emb_hbm.at[idx_v.at[j]],
                              rows_v.at[pl.ds(off, _CH)], sem_e).start()
        pltpu.make_async_copy(lin_hbm.at[idx_v.at[j]],
                              lin_v.at[pl.ds(off, _CH)], sem_l).start()
        return 0

    lax.fori_loop(0, _NCH, fire, 0)

    def drain(j, _):
        off = pl.multiple_of(j * _CH, _CH)
        pltpu.make_async_copy(emb_hbm.at[idx_v.at[j]],
                              rows_v.at[pl.ds(off, _CH)], sem_e).wait()
        pltpu.make_async_copy(lin_hbm.at[idx_v.at[j]],
                              lin_v.at[pl.ds(off, _CH)], sem_l).wait()
        return 0

    lax.fori_loop(0, _NCH, drain, 0)

    pltpu.sync_copy(rows_v, emb_out.at[pl.ds(base, _PW)])
    pltpu.sync_copy(lin_v, lin_out.at[pl.ds(base, _PW)])


def _sc_gather(emb_flat, lin_flat, idx):
    mesh = plsc.VectorSubcoreMesh(core_axis_name="c", subcore_axis_name="s")
    f = pl.kernel(
        _gather_body,
        out_type=[
            jax.ShapeDtypeStruct((_B * _F, _D), jnp.float32),
            jax.ShapeDtypeStruct((_B * _F, 1), jnp.float32),
        ],
        mesh=mesh,
        scratch_types=[
            pltpu.VMEM((_NCH, _CH), jnp.int32),
            pltpu.VMEM((_PW, _D), jnp.float32),
            pltpu.VMEM((_PW, 1), jnp.float32),
            pltpu.SemaphoreType.DMA,
            pltpu.SemaphoreType.DMA,
        ],
        compiler_params=pltpu.CompilerParams(use_tc_tiling_on_sc=False),
    )
    return f(emb_flat, lin_flat, idx)


def _dense_body(emb_ref, lin_ref, s_ref, w1_ref, b1_ref, w2_ref, b2_ref,
                w3_ref, b3_ref, wo_ref, bo_ref, out_ref):
    x = emb_ref[...]
    t = jnp.dot(x, s_ref[...], preferred_element_type=jnp.float32)
    fm = 0.5 * (jnp.sum(t * t, axis=1, keepdims=True)
                - jnp.sum(x * x, axis=1, keepdims=True))
    linear = jnp.sum(lin_ref[...], axis=1, keepdims=True)
    h = jnp.maximum(
        jnp.dot(x, w1_ref[...], preferred_element_type=jnp.float32)
        + b1_ref[...], 0.0)
    h = jnp.maximum(
        jnp.dot(h, w2_ref[...], preferred_element_type=jnp.float32)
        + b2_ref[...], 0.0)
    h = jnp.maximum(
        jnp.dot(h, w3_ref[...], preferred_element_type=jnp.float32)
        + b3_ref[...], 0.0)
    dnn = jnp.sum(h * wo_ref[...], axis=1, keepdims=True)
    out_ref[...] = jax.nn.sigmoid(linear + fm + dnn + bo_ref[...])


def _tc_dense(emb, lin, s, w1, b1, w2, b2, w3, b3, wo, bo, bm=1024):
    grid = (_B // bm,)
    full = lambda shape: pl.BlockSpec(shape, lambda i: (0, 0))
    return pl.pallas_call(
        _dense_body,
        grid=grid,
        in_specs=[
            pl.BlockSpec((bm, _F * _D), lambda i: (i, 0)),
            pl.BlockSpec((bm, _F), lambda i: (i, 0)),
            full((_F * _D, _D)),
            full((_F * _D, 256)),
            full((1, 256)),
            full((256, 128)),
            full((1, 128)),
            full((128, 64)),
            full((1, 64)),
            full((1, 64)),
            full((1, 1)),
        ],
        out_specs=pl.BlockSpec((bm, 1), lambda i: (i, 0)),
        out_shape=jax.ShapeDtypeStruct((_B, 1), jnp.float32),
    )(emb, lin, s, w1, b1, w2, b2, w3, b3, wo, bo)


def kernel(sparse_features, embed_tables, linear_tables,
           W1, b1, g1, be1, W2, b2, g2, be2, W3, b3, g3, be3, Wo, bo):
    offsets = (jnp.arange(_F, dtype=sparse_features.dtype) * _V)[None, :]
    idx = (sparse_features + offsets).astype(jnp.int32)
    idx = idx.reshape(_NW, _NCH, _CH)

    emb_rows, lin_rows = _sc_gather(
        embed_tables.reshape(_F * _V, _D),
        linear_tables.reshape(_F * _V, 1),
        idx,
    )
    emb = emb_rows.reshape(_B, _F * _D)
    lin = lin_rows.reshape(_B, _F)

    s = jnp.tile(jnp.eye(_D, dtype=jnp.float32), (_F, 1))
    inv = 1.0 / jnp.sqrt(jnp.float32(1.0 + 1e-5))
    s1, s2, s3 = g1 * inv, g2 * inv, g3 * inv
    w1f, b1f = W1 * s1[None, :], b1 * s1 + be1
    w2f, b2f = W2 * s2[None, :], b2 * s2 + be2
    w3f, b3f = W3 * s3[None, :], b3 * s3 + be3

    out = _tc_dense(emb, lin, s, w1f, b1f[None, :], w2f, b2f[None, :],
                    w3f, b3f[None, :], Wo.reshape(1, 64), bo.reshape(1, 1))
    return out[:, 0]


# speedup vs baseline: 12.4273x; 10.9452x over previous
import functools

import jax
import jax.numpy as jnp
from jax import lax
from jax.experimental import pallas as pl
from jax.experimental.pallas import tpu as pltpu
from jax.experimental.pallas import tpu_sc as plsc

_B, _F, _V, _D = 4096, 26, 100000, 16
_R = _F * _D
_NC, _NS = 2, 16
_NW = _NC * _NS
_RPW = _R // _NW
_LANES = 16


def _gather_body(emb_hbm, lin_hbm, svt_hbm, embt_out, lint_out,
                 idx_v, rows_v, vrow_v, sem, lsem):
    wid = lax.axis_index("s") * _NC + lax.axis_index("c")
    r0 = wid * _RPW

    def prep_and_fire(k, _):
        r = r0 + k
        f = r // _D
        pltpu.sync_copy(svt_hbm.at[f], vrow_v)
        base = r * _V

        def addbase(j, _):
            off = pl.multiple_of(j * _LANES, _LANES)
            idx_v[k, pl.ds(off, _LANES)] = vrow_v[pl.ds(off, _LANES)] + base
            return 0

        lax.fori_loop(0, _B // _LANES, addbase, 0)
        pltpu.make_async_copy(emb_hbm.at[idx_v.at[k]], rows_v.at[k], sem).start()
        return 0

    lax.fori_loop(0, _RPW, prep_and_fire, 0)

    def drain(k, _):
        pltpu.make_async_copy(emb_hbm.at[idx_v.at[k]], rows_v.at[k], sem).wait()
        return 0

    lax.fori_loop(0, _RPW, drain, 0)
    pltpu.sync_copy(rows_v, embt_out.at[pl.ds(r0, _RPW)])

    @pl.when(wid < _F)
    def _():
        pltpu.sync_copy(svt_hbm.at[wid], vrow_v)
        lbase = wid * _V

        def addbase2(j, _):
            off = pl.multiple_of(j * _LANES, _LANES)
            idx_v[0, pl.ds(off, _LANES)] = vrow_v[pl.ds(off, _LANES)] + lbase
            return 0

        lax.fori_loop(0, _B // _LANES, addbase2, 0)
        cp = pltpu.make_async_copy(lin_hbm.at[idx_v.at[0]], rows_v.at[0], lsem)
        cp.start()
        cp.wait()
        pltpu.sync_copy(rows_v.at[0], lint_out.at[wid])


def _sc_gather(emb_flat, lin_flat, svt):
    mesh = plsc.VectorSubcoreMesh(core_axis_name="c", subcore_axis_name="s")
    f = pl.kernel(
        _gather_body,
        out_type=[
            jax.ShapeDtypeStruct((_R, _B), jnp.float32),
            jax.ShapeDtypeStruct((_F, _B), jnp.float32),
        ],
        mesh=mesh,
        scratch_types=[
            pltpu.VMEM((_RPW, _B), jnp.int32),
            pltpu.VMEM((_RPW, _B), jnp.float32),
            pltpu.VMEM((_B,), jnp.int32),
            pltpu.SemaphoreType.DMA,
            pltpu.SemaphoreType.DMA,
        ],
        compiler_params=pltpu.CompilerParams(use_tc_tiling_on_sc=False),
    )
    return f(emb_flat, lin_flat, svt)


def _dense_body(xt_ref, lt_ref, st_ref, w1_ref, b1_ref, w2_ref, b2_ref,
                w3_ref, b3_ref, wo_ref, bo_ref, out_ref):
    x = xt_ref[...]
    t = jnp.dot(st_ref[...], x, preferred_element_type=jnp.float32)
    fm = 0.5 * (jnp.sum(t * t, axis=0, keepdims=True)
                - jnp.sum(x * x, axis=0, keepdims=True))
    linear = jnp.sum(lt_ref[...], axis=0, keepdims=True)
    h = jnp.maximum(
        jnp.dot(w1_ref[...], x, preferred_element_type=jnp.float32)
        + b1_ref[...], 0.0)
    h = jnp.maximum(
        jnp.dot(w2_ref[...], h, preferred_element_type=jnp.float32)
        + b2_ref[...], 0.0)
    h = jnp.maximum(
        jnp.dot(w3_ref[...], h, preferred_element_type=jnp.float32)
        + b3_ref[...], 0.0)
    dnn = jnp.sum(h * wo_ref[...], axis=0, keepdims=True)
    out_ref[...] = jax.nn.sigmoid(linear + fm + dnn + bo_ref[...])


def _tc_dense(xt, lt, st, w1t, b1, w2t, b2, w3t, b3, wo, bo, bn=1024):
    grid = (_B // bn,)
    full = lambda shape: pl.BlockSpec(shape, lambda i: (0, 0))
    return pl.pallas_call(
        _dense_body,
        grid=grid,
        in_specs=[
            pl.BlockSpec((_R, bn), lambda i: (0, i)),
            pl.BlockSpec((_F, bn), lambda i: (0, i)),
            full((_D, _R)),
            full((256, _R)),
            full((256, 1)),
            full((128, 256)),
            full((128, 1)),
            full((64, 128)),
            full((64, 1)),
            full((64, 1)),
            full((1, 1)),
        ],
        out_specs=pl.BlockSpec((1, bn), lambda i: (0, i)),
        out_shape=jax.ShapeDtypeStruct((1, _B), jnp.float32),
    )(xt, lt, st, w1t, b1, w2t, b2, w3t, b3, wo, bo)


def kernel(sparse_features, embed_tables, linear_tables,
           W1, b1, g1, be1, W2, b2, g2, be2, W3, b3, g3, be3, Wo, bo):
    emb_flat = jnp.transpose(embed_tables, (0, 2, 1)).reshape(_R * _V)
    lin_flat = jnp.transpose(linear_tables, (0, 2, 1)).reshape(_F * _V)
    svt = jnp.transpose(sparse_features).astype(jnp.int32)

    xt, lt = _sc_gather(emb_flat, lin_flat, svt)

    st = jnp.tile(jnp.eye(_D, dtype=jnp.float32), (1, _F))
    inv = 1.0 / jnp.sqrt(jnp.float32(1.0 + 1e-5))
    s1, s2, s3 = g1 * inv, g2 * inv, g3 * inv
    w1t = (W1 * s1[None, :]).T
    b1f = (b1 * s1 + be1)[:, None]
    w2t = (W2 * s2[None, :]).T
    b2f = (b2 * s2 + be2)[:, None]
    w3t = (W3 * s3[None, :]).T
    b3f = (b3 * s3 + be3)[:, None]

    out = _tc_dense(xt, lt, st, w1t, b1f, w2t, b2f, w3t, b3f,
                    Wo, bo.reshape(1, 1))
    return out.reshape(_B)


# speedup vs baseline: 12.5931x; 1.0133x over previous
import functools

import jax
import jax.numpy as jnp
from jax import lax
from jax.experimental import pallas as pl
from jax.experimental.pallas import tpu as pltpu
from jax.experimental.pallas import tpu_sc as plsc

_B, _F, _V, _D = 4096, 26, 100000, 16
_R = _F * _D
_NC, _NS = 2, 16
_NW = _NC * _NS
_RPW = _R // _NW
_LANES = 16


def _gather_body(emb_hbm, lin_hbm, svt_hbm, embt_out, lint_out,
                 vrows_v, rows_v, lrow_v, sem, lsem):
    wid = lax.axis_index("s") * _NC + lax.axis_index("c")
    r0 = wid * _RPW
    f0 = r0 // _D
    pltpu.sync_copy(svt_hbm.at[f0], vrows_v.at[0])
    f1 = (r0 + _RPW - 1) // _D
    pltpu.sync_copy(svt_hbm.at[f1], vrows_v.at[1])

    @pl.when(wid < _F)
    def _():
        pltpu.sync_copy(svt_hbm.at[wid], vrows_v.at[2])
        pltpu.make_async_copy(
            lin_hbm.at[pl.ds(wid * _V, _V)].at[vrows_v.at[2]],
            lrow_v, lsem).start()

    def fire(k, _):
        r = r0 + k
        sel = r // _D - f0
        pltpu.make_async_copy(
            emb_hbm.at[pl.ds(r * _V, _V)].at[vrows_v.at[sel]],
            rows_v.at[k], sem).start()
        return 0

    lax.fori_loop(0, _RPW, fire, 0)

    def drain(k, _):
        r = r0 + k
        sel = r // _D - f0
        pltpu.make_async_copy(
            emb_hbm.at[pl.ds(r * _V, _V)].at[vrows_v.at[sel]],
            rows_v.at[k], sem).wait()
        return 0

    lax.fori_loop(0, _RPW, drain, 0)
    pltpu.sync_copy(rows_v, embt_out.at[pl.ds(r0, _RPW)])

    @pl.when(wid < _F)
    def _():
        pltpu.make_async_copy(
            lin_hbm.at[pl.ds(wid * _V, _V)].at[vrows_v.at[2]],
            lrow_v, lsem).wait()
        pltpu.sync_copy(lrow_v, lint_out.at[wid])


def _sc_gather(emb_flat, lin_flat, svt):
    mesh = plsc.VectorSubcoreMesh(core_axis_name="c", subcore_axis_name="s")
    f = pl.kernel(
        _gather_body,
        out_type=[
            jax.ShapeDtypeStruct((_R, _B), jnp.float32),
            jax.ShapeDtypeStruct((_F, _B), jnp.float32),
        ],
        mesh=mesh,
        scratch_types=[
            pltpu.VMEM((3, _B), jnp.int32),
            pltpu.VMEM((_RPW, _B), jnp.float32),
            pltpu.VMEM((_B,), jnp.float32),
            pltpu.SemaphoreType.DMA,
            pltpu.SemaphoreType.DMA,
        ],
        compiler_params=pltpu.CompilerParams(use_tc_tiling_on_sc=False),
    )
    return f(emb_flat, lin_flat, svt)


def _dense_body(xt_ref, lt_ref, st_ref, w1_ref, b1_ref, w2_ref, b2_ref,
                w3_ref, b3_ref, wo_ref, bo_ref, out_ref):
    x = xt_ref[...]
    t = jnp.dot(st_ref[...], x, preferred_element_type=jnp.float32)
    fm = 0.5 * (jnp.sum(t * t, axis=0, keepdims=True)
                - jnp.sum(x * x, axis=0, keepdims=True))
    linear = jnp.sum(lt_ref[...], axis=0, keepdims=True)
    h = jnp.maximum(
        jnp.dot(w1_ref[...], x, preferred_element_type=jnp.float32)
        + b1_ref[...], 0.0)
    h = jnp.maximum(
        jnp.dot(w2_ref[...], h, preferred_element_type=jnp.float32)
        + b2_ref[...], 0.0)
    h = jnp.maximum(
        jnp.dot(w3_ref[...], h, preferred_element_type=jnp.float32)
        + b3_ref[...], 0.0)
    dnn = jnp.sum(h * wo_ref[...], axis=0, keepdims=True)
    out_ref[...] = jax.nn.sigmoid(linear + fm + dnn + bo_ref[...])


def _tc_dense(xt, lt, st, w1t, b1, w2t, b2, w3t, b3, wo, bo, bn=1024):
    grid = (_B // bn,)
    full = lambda shape: pl.BlockSpec(shape, lambda i: (0, 0))
    return pl.pallas_call(
        _dense_body,
        grid=grid,
        in_specs=[
            pl.BlockSpec((_R, bn), lambda i: (0, i)),
            pl.BlockSpec((_F, bn), lambda i: (0, i)),
            full((_D, _R)),
            full((256, _R)),
            full((256, 1)),
            full((128, 256)),
            full((128, 1)),
            full((64, 128)),
            full((64, 1)),
            full((64, 1)),
            full((1, 1)),
        ],
        out_specs=pl.BlockSpec((1, bn), lambda i: (0, i)),
        out_shape=jax.ShapeDtypeStruct((1, _B), jnp.float32),
    )(xt, lt, st, w1t, b1, w2t, b2, w3t, b3, wo, bo)


def kernel(sparse_features, embed_tables, linear_tables,
           W1, b1, g1, be1, W2, b2, g2, be2, W3, b3, g3, be3, Wo, bo):
    emb_flat = jnp.transpose(embed_tables, (0, 2, 1)).reshape(_R * _V)
    lin_flat = jnp.transpose(linear_tables, (0, 2, 1)).reshape(_F * _V)
    svt = jnp.transpose(sparse_features).astype(jnp.int32)

    xt, lt = _sc_gather(emb_flat, lin_flat, svt)

    st = jnp.tile(jnp.eye(_D, dtype=jnp.float32), (1, _F))
    inv = 1.0 / jnp.sqrt(jnp.float32(1.0 + 1e-5))
    s1, s2, s3 = g1 * inv, g2 * inv, g3 * inv
    w1t = (W1 * s1[None, :]).T
    b1f = (b1 * s1 + be1)[:, None]
    w2t = (W2 * s2[None, :]).T
    b2f = (b2 * s2 + be2)[:, None]
    w3t = (W3 * s3[None, :]).T
    b3f = (b3 * s3 + be3)[:, None]

    out = _tc_dense(xt, lt, st, w1t, b1f, w2t, b2f, w3t, b3f,
                    Wo, bo.reshape(1, 1))
    return out.reshape(_B)


# speedup vs baseline: 14.5224x; 1.1532x over previous
import functools

import jax
import jax.numpy as jnp
from jax import lax
from jax.experimental import pallas as pl
from jax.experimental.pallas import tpu as pltpu
from jax.experimental.pallas import tpu_sc as plsc

_B, _F, _V, _D = 4096, 26, 100000, 16
_R = _F * _D
_NC, _NS = 2, 16
_NW = _NC * _NS
_RPW = _R // _NW
_LANES = 16


def _gather_body(emb_hbm, svt_hbm, embt_out, vrows_v, rows_v, sem):
    wid = lax.axis_index("s") * _NC + lax.axis_index("c")
    r0 = wid * _RPW
    f0 = r0 // _D
    pltpu.sync_copy(svt_hbm.at[f0], vrows_v.at[0])
    f1 = (r0 + _RPW - 1) // _D
    pltpu.sync_copy(svt_hbm.at[f1], vrows_v.at[1])

    def fire(k, _):
        r = r0 + k
        sel = r // _D - f0
        pltpu.make_async_copy(
            emb_hbm.at[pl.ds(r * _V, _V)].at[vrows_v.at[sel]],
            rows_v.at[k], sem).start()
        return 0

    lax.fori_loop(0, _RPW, fire, 0)

    def drain(k, _):
        r = r0 + k
        sel = r // _D - f0
        pltpu.make_async_copy(
            emb_hbm.at[pl.ds(r * _V, _V)].at[vrows_v.at[sel]],
            rows_v.at[k], sem).wait()
        return 0

    lax.fori_loop(0, _RPW, drain, 0)
    pltpu.sync_copy(rows_v, embt_out.at[pl.ds(r0, _RPW)])


def _lin_gather_body(lin_hbm, svt_hbm, lint_out, vrow_v, lrow_v, sem):
    wid = lax.axis_index("s") * _NC + lax.axis_index("c")

    @pl.when(wid < _F)
    def _():
        pltpu.sync_copy(svt_hbm.at[wid], vrow_v)
        cp = pltpu.make_async_copy(
            lin_hbm.at[pl.ds(wid * _V, _V)].at[vrow_v], lrow_v, sem)
        cp.start()
        cp.wait()
        pltpu.sync_copy(lrow_v, lint_out.at[wid])


def _sc_gather(emb_flat, svt):
    mesh = plsc.VectorSubcoreMesh(core_axis_name="c", subcore_axis_name="s")
    f = pl.kernel(
        _gather_body,
        out_type=jax.ShapeDtypeStruct((_R, _B), jnp.float32),
        mesh=mesh,
        scratch_types=[
            pltpu.VMEM((2, _B), jnp.int32),
            pltpu.VMEM((_RPW, _B), jnp.float32),
            pltpu.SemaphoreType.DMA,
        ],
        compiler_params=pltpu.CompilerParams(use_tc_tiling_on_sc=False),
    )
    return f(emb_flat, svt)


def _sc_lin_gather(lin_flat, svt):
    mesh = plsc.VectorSubcoreMesh(core_axis_name="c", subcore_axis_name="s")
    f = pl.kernel(
        _lin_gather_body,
        out_type=jax.ShapeDtypeStruct((_F, _B), jnp.float32),
        mesh=mesh,
        scratch_types=[
            pltpu.VMEM((_B,), jnp.int32),
            pltpu.VMEM((_B,), jnp.float32),
            pltpu.SemaphoreType.DMA,
        ],
        compiler_params=pltpu.CompilerParams(use_tc_tiling_on_sc=False),
    )
    return f(lin_flat, svt)


def _dense_body(xt_ref, lt_ref, st_ref, w1_ref, b1_ref, w2_ref, b2_ref,
                w3_ref, b3_ref, wo_ref, bo_ref, out_ref):
    x = xt_ref[...]
    t = jnp.dot(st_ref[...], x, preferred_element_type=jnp.float32)
    fm = 0.5 * (jnp.sum(t * t, axis=0, keepdims=True)
                - jnp.sum(x * x, axis=0, keepdims=True))
    linear = jnp.sum(lt_ref[...], axis=0, keepdims=True)
    h = jnp.maximum(
        jnp.dot(w1_ref[...], x, preferred_element_type=jnp.float32)
        + b1_ref[...], 0.0)
    h = jnp.maximum(
        jnp.dot(w2_ref[...], h, preferred_element_type=jnp.float32)
        + b2_ref[...], 0.0)
    h = jnp.maximum(
        jnp.dot(w3_ref[...], h, preferred_element_type=jnp.float32)
        + b3_ref[...], 0.0)
    dnn = jnp.sum(h * wo_ref[...], axis=0, keepdims=True)
    out_ref[...] = jax.nn.sigmoid(linear + fm + dnn + bo_ref[...])


def _tc_dense(xt, lt, st, w1t, b1, w2t, b2, w3t, b3, wo, bo, bn=1024):
    grid = (_B // bn,)
    full = lambda shape: pl.BlockSpec(shape, lambda i: (0, 0))
    return pl.pallas_call(
        _dense_body,
        grid=grid,
        in_specs=[
            pl.BlockSpec((_R, bn), lambda i: (0, i)),
            pl.BlockSpec((_F, bn), lambda i: (0, i)),
            full((_D, _R)),
            full((256, _R)),
            full((256, 1)),
            full((128, 256)),
            full((128, 1)),
            full((64, 128)),
            full((64, 1)),
            full((64, 1)),
            full((1, 1)),
        ],
        out_specs=pl.BlockSpec((1, bn), lambda i: (0, i)),
        out_shape=jax.ShapeDtypeStruct((1, _B), jnp.float32),
    )(xt, lt, st, w1t, b1, w2t, b2, w3t, b3, wo, bo)


def kernel(sparse_features, embed_tables, linear_tables,
           W1, b1, g1, be1, W2, b2, g2, be2, W3, b3, g3, be3, Wo, bo):
    emb_flat = jnp.transpose(embed_tables, (0, 2, 1)).reshape(_R * _V)
    lin_flat = linear_tables.reshape(_F * _V)
    svt = jnp.transpose(sparse_features).astype(jnp.int32)

    xt = _sc_gather(emb_flat, svt)
    lt = _sc_lin_gather(lin_flat, svt)

    st = jnp.tile(jnp.eye(_D, dtype=jnp.float32), (1, _F))
    inv = 1.0 / jnp.sqrt(jnp.float32(1.0 + 1e-5))
    s1, s2, s3 = g1 * inv, g2 * inv, g3 * inv
    w1t = (W1 * s1[None, :]).T
    b1f = (b1 * s1 + be1)[:, None]
    w2t = (W2 * s2[None, :]).T
    b2f = (b2 * s2 + be2)[:, None]
    w3t = (W3 * s3[None, :]).T
    b3f = (b3 * s3 + be3)[:, None]

    out = _tc_dense(xt, lt, st, w1t, b1f, w2t, b2f, w3t, b3f,
                    Wo, bo.reshape(1, 1))
    return out.reshape(_B)
